# baseline (speedup 1.0000x reference)
import functools

import jax
import jax.numpy as jnp
from jax import lax
from jax.experimental import pallas as pl
from jax.experimental.pallas import tpu as pltpu
from jax.experimental.pallas import tpu_sc as plsc

B = 4
S = 2048
D = 2048
G = B * S

_info = plsc.get_sparse_core_info()
_NC = _info.num_cores
_NS = _info.num_subcores
_L = _info.num_lanes
_NW = _NC * _NS
_ROWS_PER_W = G // _NW
_K = 16
_NCHUNK = _ROWS_PER_W // _K

_mesh = plsc.VectorSubcoreMesh(core_axis_name="c", subcore_axis_name="s")


@functools.partial(
    pl.kernel,
    mesh=_mesh,
    out_type=jax.ShapeDtypeStruct((G, D), jnp.float32),
    scratch_types=[
        pltpu.VMEM((_K,), jnp.int32),
        pltpu.VMEM((_K, D), jnp.float32),
        pltpu.VMEM((_K, D), jnp.float32),
        pltpu.SemaphoreType.DMA,
    ],
)
def _embed(x_hbm, table_hbm, pe_hbm, out_hbm, idx_v, rows_v, pe_v, sem):
    wid = lax.axis_index("s") * _NC + lax.axis_index("c")
    base = wid * _ROWS_PER_W
    for c in range(_NCHUNK):
        g0 = pl.multiple_of(base + c * _K, _K)
        s0 = pl.multiple_of(lax.rem(g0, S), _K)
        pltpu.sync_copy(x_hbm.at[pl.ds(g0, _K)], idx_v)
        gather = pltpu.async_copy(table_hbm.at[idx_v], rows_v, sem)
        pltpu.sync_copy(pe_hbm.at[pl.ds(s0, _K)], pe_v)
        gather.wait()
        for r in range(_K):
            def _add(j, _, r=r):
                sl = pl.ds(j * _L, _L)
                rows_v[r, sl] = rows_v[r, sl] + pe_v[r, sl]
                return 0
            lax.fori_loop(0, D // _L, _add, 0)
        pltpu.sync_copy(rows_v, out_hbm.at[pl.ds(g0, _K)])


def kernel(x, token_embedding, pe):
    x_flat = x.reshape(G).astype(jnp.int32)
    out = _embed(x_flat, token_embedding, pe)
    return out.reshape(B, S, D)


# speedup vs baseline: 2.7808x; 2.7808x over previous
import functools

import jax
import jax.numpy as jnp
from jax import lax
from jax.experimental import pallas as pl
from jax.experimental.pallas import tpu as pltpu
from jax.experimental.pallas import tpu_sc as plsc

B = 4
S = 2048
D = 2048
G = B * S

_info = plsc.get_sparse_core_info()
_NC = _info.num_cores
_NS = _info.num_subcores
_L = _info.num_lanes
_NW = _NC * _NS
_SPW = S // _NW
_K = 16
_NCH = _SPW // _K
_T = _NCH * B

_mesh = plsc.VectorSubcoreMesh(core_axis_name="c", subcore_axis_name="s")


@functools.partial(
    pl.kernel,
    mesh=_mesh,
    out_type=jax.ShapeDtypeStruct((G, D), jnp.float32),
    scratch_types=[
        pltpu.VMEM((B, _SPW), jnp.int32),
        pltpu.VMEM((_K, D), jnp.float32),
        pltpu.VMEM((_K, D), jnp.float32),
        pltpu.VMEM((_K, D), jnp.float32),
        pltpu.SemaphoreType.DMA,
        pltpu.SemaphoreType.DMA,
        pltpu.SemaphoreType.DMA,
        pltpu.SemaphoreType.DMA,
    ],
)
def _embed(x_hbm, table_hbm, pe_hbm, out_hbm, idx_all, rows0, rows1,
           pe_v, sg0, sg1, sw0, sw1):
    wid = lax.axis_index("s") * _NC + lax.axis_index("c")
    s_base = pl.multiple_of(wid * _SPW, _SPW)

    for b in range(B):
        pltpu.sync_copy(x_hbm.at[b, pl.ds(s_base, _SPW)], idx_all.at[b])

    rows = (rows0, rows1)
    sg = (sg0, sg1)
    sw = (sw0, sw1)

    def issue_gather(t):
        c, b = divmod(t, B)
        return pltpu.async_copy(
            table_hbm.at[idx_all.at[b, pl.ds(c * _K, _K)]], rows[t % 2],
            sg[t % 2])

    gather_h = {0: None}
    write_h = {}
    pltpu.sync_copy(pe_hbm.at[pl.ds(s_base, _K)], pe_v)
    gather_h[0] = issue_gather(0)

    for t in range(_T):
        c, b = divmod(t, B)
        if t + 1 < _T:
            if t - 1 >= 0:
                write_h[t - 1].wait()
            gather_h[t + 1] = issue_gather(t + 1)
        gather_h[t].wait()

        buf = rows[t % 2]

        @plsc.parallel_loop(0, D, step=_L)
        def _add(i):
            for r in range(_K):
                plsc.addupdate(buf.at[r, pl.ds(i, _L)],
                               pe_v[r, pl.ds(i, _L)])

        g0 = pl.multiple_of(b * S + s_base + c * _K, _K)
        write_h[t] = pltpu.async_copy(buf, out_hbm.at[pl.ds(g0, _K)],
                                      sw[t % 2])
        if b == B - 1 and c + 1 < _NCH:
            pltpu.sync_copy(pe_hbm.at[pl.ds(s_base + (c + 1) * _K, _K)],
                            pe_v)

    write_h[_T - 2].wait()
    write_h[_T - 1].wait()


def kernel(x, token_embedding, pe):
    out = _embed(x.astype(jnp.int32), token_embedding, pe)
    return out.reshape(B, S, D)


# speedup vs baseline: 2.9524x; 1.0617x over previous
import functools

import jax
import jax.numpy as jnp
from jax import lax
from jax.experimental import pallas as pl
from jax.experimental.pallas import tpu as pltpu
from jax.experimental.pallas import tpu_sc as plsc

B = 4
S = 2048
D = 2048
G = B * S

_info = plsc.get_sparse_core_info()
_NC = _info.num_cores
_NS = _info.num_subcores
_L = _info.num_lanes
_NW = _NC * _NS
_SPW = S // _NW
_K = 8
_NCH = _SPW // _K
_T = _NCH * B
_NBUF = 4

_mesh = plsc.VectorSubcoreMesh(core_axis_name="c", subcore_axis_name="s")


@functools.partial(
    pl.kernel,
    mesh=_mesh,
    out_type=jax.ShapeDtypeStruct((G, D), jnp.float32),
    scratch_types=[
        pltpu.VMEM((B, _SPW), jnp.int32),
        pltpu.VMEM((_NBUF, _K, D), jnp.float32),
        pltpu.VMEM((2, _K, D), jnp.float32),
        pltpu.SemaphoreType.DMA((_NBUF,)),
        pltpu.SemaphoreType.DMA((_NBUF,)),
        pltpu.SemaphoreType.DMA((2,)),
    ],
)
def _embed(x_hbm, table_hbm, pe_hbm, out_hbm, idx_all, rows, pe_v,
           sg, sw, sp):
    wid = lax.axis_index("s") * _NC + lax.axis_index("c")
    s_base = pl.multiple_of(wid * _SPW, _SPW)

    for b in range(B):
        pltpu.sync_copy(x_hbm.at[b, pl.ds(s_base, _SPW)], idx_all.at[b])

    def issue_gather(t):
        c, b = divmod(t, B)
        slot = t % _NBUF
        return pltpu.async_copy(
            table_hbm.at[idx_all.at[b, pl.ds(c * _K, _K)]],
            rows.at[slot], sg.at[slot])

    def issue_pe(c):
        return pltpu.async_copy(
            pe_hbm.at[pl.ds(s_base + c * _K, _K)],
            pe_v.at[c % 2], sp.at[c % 2])

    pe_h = {0: issue_pe(0), 1: issue_pe(1)}
    gather_h = {}
    write_h = {}
    for tp in range(_NBUF - 1):
        gather_h[tp] = issue_gather(tp)

    for t in range(_T):
        c, b = divmod(t, B)
        tp = t + _NBUF - 1
        if tp < _T:
            if tp - _NBUF >= 0:
                write_h[tp - _NBUF].wait()
            gather_h[tp] = issue_gather(tp)
        if b == 0:
            pe_h[c].wait()
        gather_h[t].wait()

        slot = t % _NBUF
        pc = c % 2

        @plsc.parallel_loop(0, D, step=_L)
        def _add(i):
            for r in range(_K):
                plsc.addupdate(rows.at[slot, r, pl.ds(i, _L)],
                               pe_v[pc, r, pl.ds(i, _L)])

        g0 = pl.multiple_of(b * S + s_base + c * _K, _K)
        write_h[t] = pltpu.async_copy(rows.at[slot],
                                      out_hbm.at[pl.ds(g0, _K)],
                                      sw.at[slot])
        if b == B - 1 and c + 2 < _NCH:
            pe_h[c + 2] = issue_pe(c + 2)

    for t in range(_T - _NBUF, _T):
        write_h[t].wait()


def kernel(x, token_embedding, pe):
    out = _embed(x.astype(jnp.int32), token_embedding, pe)
    return out.reshape(B, S, D)
